# baseline (speedup 1.0000x reference)
import functools

import jax
import jax.numpy as jnp
from jax import lax
from jax.experimental import pallas as pl
from jax.experimental.pallas import tpu as pltpu
from jax.experimental.pallas import tpu_sc as plsc

CAP = 65536
X_DIM = 128
Y0, Y1 = 32, 32
Y_FLAT = Y0 * Y1

NC = 2
NS = 16
NW = NC * NS

CAM_TROWS = CAP // NW
ST_TROWS = CAP // NW

SR_C = 16
NBUF_C = 4
LAG_C = 2
CAM_STRIPES = CAM_TROWS // SR_C

SR_S = 128
NBUF_S = 2
ST_STRIPES = ST_TROWS // SR_S

IT_CHUNK = 16384
IT_STRIPES = CAP // IT_CHUNK


def _sc_body(pos16_h, cnt16_h, srow_h, crow_h, sb_h, cb_h, it_h,
             sb_o, cb_o, it_o,
             cam_buf, st_buf, it_buf, srow_v, crow_v, posv, cntv,
             sem_ci, sem_co, sem_si, sem_so, sem_it, sem_row):
    wid = lax.axis_index("s") * NC + lax.axis_index("c")

    pltpu.async_copy(pos16_h, posv, sem_row).wait()
    pltpu.async_copy(cnt16_h, cntv, sem_row).wait()
    pos = posv[...][0]
    cnt = cntv[...][0]

    cam_base = wid * CAM_TROWS
    st_base = wid * ST_TROWS

    ins = {}
    outs = {}

    def cam_in(s):
        b = s % NBUF_C
        ins[s] = pltpu.async_copy(
            cb_h.at[pl.ds(cam_base + s * SR_C, SR_C)], cam_buf.at[b],
            sem_ci.at[b])

    def cam_out(s):
        b = s % NBUF_C
        outs[s] = pltpu.async_copy(
            cam_buf.at[b], cb_o.at[pl.ds(cam_base + s * SR_C, SR_C)],
            sem_co.at[b])

    for s in range(min(NBUF_C, CAM_STRIPES)):
        cam_in(s)
    for s in range(CAM_STRIPES):
        ins[s].wait()
        cam_out(s)
        t = s - LAG_C
        if t >= 0 and t + NBUF_C < CAM_STRIPES:
            outs[t].wait()
            cam_in(t + NBUF_C)
    for s in range(CAM_STRIPES):
        if s + NBUF_C >= CAM_STRIPES or s < LAG_C:
            pass
    waited = set(range(0, CAM_STRIPES - NBUF_C))
    for s in range(CAM_STRIPES):
        if s not in waited:
            outs[s].wait()

    @pl.when(wid == pos // CAM_TROWS)
    def _cam_row():
        pltpu.async_copy(crow_h, crow_v, sem_row).wait()
        pltpu.async_copy(crow_v, cb_o.at[pl.ds(pos, 1)], sem_row).wait()

    sins = {}
    souts = {}

    def st_in(s):
        b = s % NBUF_S
        sins[s] = pltpu.async_copy(
            sb_h.at[pl.ds(st_base + s * SR_S, SR_S)], st_buf.at[b],
            sem_si.at[b])

    def st_out(s):
        b = s % NBUF_S
        souts[s] = pltpu.async_copy(
            st_buf.at[b], sb_o.at[pl.ds(st_base + s * SR_S, SR_S)],
            sem_so.at[b])

    for s in range(min(NBUF_S, ST_STRIPES)):
        st_in(s)
    for s in range(ST_STRIPES):
        sins[s].wait()
        st_out(s)
        t = s - 1
        if t >= 0 and t + NBUF_S < ST_STRIPES:
            souts[t].wait()
            st_in(t + NBUF_S)
    swaited = set(range(0, ST_STRIPES - NBUF_S))
    for s in range(ST_STRIPES):
        if s not in swaited:
            souts[s].wait()

    @pl.when(wid == pos // ST_TROWS)
    def _st_row():
        pltpu.async_copy(srow_h, srow_v, sem_row).wait()
        pltpu.async_copy(srow_v, sb_o.at[pl.ds(pos, 1)], sem_row).wait()

    @pl.when(wid == 0)
    def _iter():
        for s in range(IT_STRIPES):
            base = s * IT_CHUNK
            pltpu.async_copy(it_h.at[pl.ds(base, IT_CHUNK)], it_buf,
                             sem_it).wait()

            @pl.when((pos >= base) & (pos < base + IT_CHUNK))
            def _patch(base=base):
                local = pos - base
                off = (local // 16) * 16
                lane = local - off
                v = it_buf[pl.ds(off, 16)]
                w = jnp.where(lax.iota(jnp.int32, 16) == lane, cnt, v)
                it_buf[pl.ds(off, 16)] = w

            pltpu.async_copy(it_buf, it_o.at[pl.ds(base, IT_CHUNK)],
                             sem_it).wait()


@functools.partial(jax.jit, static_argnums=())
def sc_push(state_buffer, cam2d, iter_buffer, pos16, cnt16, srow, crow):
    mesh = plsc.VectorSubcoreMesh(core_axis_name="c", subcore_axis_name="s")
    f = pl.kernel(
        _sc_body,
        out_type=[
            jax.ShapeDtypeStruct((CAP, X_DIM), jnp.float32),
            jax.ShapeDtypeStruct((CAP, Y_FLAT), jnp.float32),
            jax.ShapeDtypeStruct((CAP,), jnp.int32),
        ],
        mesh=mesh,
        scratch_types=[
            pltpu.VMEM((NBUF_C, SR_C, Y_FLAT), jnp.float32),
            pltpu.VMEM((NBUF_S, SR_S, X_DIM), jnp.float32),
            pltpu.VMEM((IT_CHUNK,), jnp.int32),
            pltpu.VMEM((1, X_DIM), jnp.float32),
            pltpu.VMEM((1, Y_FLAT), jnp.float32),
            pltpu.VMEM((16,), jnp.int32),
            pltpu.VMEM((16,), jnp.int32),
            pltpu.SemaphoreType.DMA((NBUF_C,)),
            pltpu.SemaphoreType.DMA((NBUF_C,)),
            pltpu.SemaphoreType.DMA((NBUF_S,)),
            pltpu.SemaphoreType.DMA((NBUF_S,)),
            pltpu.SemaphoreType.DMA,
            pltpu.SemaphoreType.DMA,
        ],
    )
    return f(pos16, cnt16, srow, crow, state_buffer, cam2d, iter_buffer)


def kernel(state_buffer, cam_data_buffer, iter_buffer, position, state,
           cam_data, count):
    pos16 = jnp.broadcast_to(position, (16,)).astype(jnp.int32)
    cnt16 = jnp.broadcast_to(count, (16,)).astype(jnp.int32)
    srow = state.reshape(1, X_DIM)
    crow = cam_data.reshape(1, Y_FLAT)
    cam2d = cam_data_buffer.reshape(CAP, Y_FLAT)

    out_sb, out_cb, out_it = sc_push(state_buffer, cam2d, iter_buffer,
                                     pos16, cnt16, srow, crow)

    new_position = jnp.remainder(position + 1, CAP)
    full_buffer = (position + 1) == CAP
    return (out_sb, out_cb.reshape(CAP, Y0, Y1), out_it, new_position,
            full_buffer)
